# baseline (speedup 1.0000x reference)
import functools

import jax
import jax.numpy as jnp
from jax import lax
from jax.experimental import pallas as pl
from jax.experimental.pallas import tpu as pltpu
from jax.experimental.pallas import tpu_sc as plsc

NUM_NODES = 1000000
EMBED_DIM = 64
BATCH = 16384
SPLIT = 368640
DENSE_ROWS = (NUM_NODES - SPLIT) // 2

_info = plsc.get_sparse_core_info()
_NC, _NS, _L = _info.num_cores, _info.num_subcores, _info.num_lanes
_NW = _NC * _NS
_B_PER_W = BATCH // _NW
_CHUNK = 256


@functools.partial(
    pl.kernel,
    mesh=plsc.VectorSubcoreMesh(core_axis_name="c", subcore_axis_name="s"),
    out_type=jax.ShapeDtypeStruct((BATCH, EMBED_DIM), jnp.float32),
    scratch_types=[
        pltpu.VMEM((_B_PER_W,), jnp.int32),
        pltpu.VMEM((_B_PER_W, EMBED_DIM), jnp.float32),
        pltpu.SemaphoreType.DMA,
    ],
    compiler_params=pltpu.CompilerParams(needs_layout_passes=False),
)
def _gather_low(table_hbm, idx_hbm, out_hbm, idx_v, rows_v, sem):
    wid = lax.axis_index("s") * _NC + lax.axis_index("c")
    base = wid * _B_PER_W
    pltpu.sync_copy(idx_hbm.at[pl.ds(base, _B_PER_W)], idx_v)

    def fire(g, n_own):
        vec = idx_v[pl.ds(g * _L, _L)]
        cnt = jnp.sum(jnp.where(vec < SPLIT, 1, 0).astype(jnp.int32))
        for t in range(_L):
            i = vec[t]

            @pl.when(i < SPLIT)
            def _():
                pltpu.make_async_copy(
                    table_hbm.at[i], rows_v.at[g * _L + t], sem
                ).start()

        return n_own + cnt

    n_own = lax.fori_loop(0, _B_PER_W // _L, fire, jnp.int32(0))

    def drain(_, carry):
        pltpu.make_async_copy(
            table_hbm.at[0], rows_v.at[0], sem
        ).wait()
        return carry

    lax.fori_loop(0, n_own, drain, 0)
    pltpu.sync_copy(rows_v, out_hbm.at[pl.ds(base, _B_PER_W)])


@functools.partial(
    pl.kernel,
    mesh=plsc.VectorSubcoreMesh(core_axis_name="c", subcore_axis_name="s"),
    out_type=jax.ShapeDtypeStruct((BATCH, EMBED_DIM), jnp.float32),
    scratch_types=[
        pltpu.VMEM((_B_PER_W,), jnp.int32),
        pltpu.VMEM((_B_PER_W,), jnp.int32),
        pltpu.VMEM((_B_PER_W,), jnp.int32),
        pltpu.VMEM((_CHUNK, 2 * EMBED_DIM), jnp.float32),
        pltpu.VMEM((_B_PER_W, EMBED_DIM), jnp.float32),
        pltpu.SemaphoreType.DMA,
    ],
    compiler_params=pltpu.CompilerParams(needs_layout_passes=False),
)
def _gather_high(
    dense_hbm, idx_hbm, out_hbm, idx_v, k_v, h_v, slab_v, rows_v, sem
):
    wid = lax.axis_index("s") * _NC + lax.axis_index("c")
    base = wid * _B_PER_W
    pltpu.sync_copy(idx_hbm.at[pl.ds(base, _B_PER_W)], idx_v)

    for g in range(_B_PER_W // _L):
        sl = pl.ds(g * _L, _L)
        rel = jnp.maximum(idx_v[sl] - SPLIT, 0)
        k_v[sl] = lax.shift_right_logical(rel, 1)
        h_v[sl] = lax.bitwise_and(rel, 1)

    for c in range(_B_PER_W // _CHUNK):
        cbase = c * _CHUNK
        pltpu.async_copy(
            dense_hbm.at[k_v.at[pl.ds(cbase, _CHUNK)]], slab_v, sem
        ).wait()

        for g in range(_CHUNK // _L):
            rbase = cbase + g * _L
            lrow = lax.iota(jnp.int32, _L) + g * _L
            grow = lax.iota(jnp.int32, _L) + rbase
            hvec = h_v[pl.ds(rbase, _L)]
            own = idx_v[pl.ds(rbase, _L)] >= SPLIT
            woff = hvec * EMBED_DIM

            def col_body(t, carry, lrow=lrow, grow=grow, woff=woff, own=own):
                tcol = jnp.full((_L,), 0, jnp.int32) + t
                vals = plsc.load_gather(slab_v, [lrow, woff + tcol])
                plsc.store_scatter(rows_v, [grow, tcol], vals, mask=own)
                return carry

            lax.fori_loop(0, EMBED_DIM, col_body, 0)

    pltpu.sync_copy(rows_v, out_hbm.at[pl.ds(base, _B_PER_W)])


def kernel(node_indices, embeddings):
    idx = node_indices.astype(jnp.int32)
    dense = jnp.reshape(
        lax.slice(embeddings, (SPLIT, 0), (NUM_NODES, EMBED_DIM)),
        (DENSE_ROWS, 2 * EMBED_DIM),
    )
    out_low = _gather_low(embeddings, idx)
    out_high = _gather_high(dense, idx)
    return jnp.where((idx < SPLIT)[:, None], out_low, out_high)


# speedup vs baseline: 2.8924x; 2.8924x over previous
import functools

import jax
import jax.numpy as jnp
from jax import lax
from jax.experimental import pallas as pl
from jax.experimental.pallas import tpu as pltpu
from jax.experimental.pallas import tpu_sc as plsc

NUM_NODES = 1000000
EMBED_DIM = 64
BATCH = 16384

_info = plsc.get_sparse_core_info()
_NC, _NS, _L = _info.num_cores, _info.num_subcores, _info.num_lanes
_NW = _NC * _NS
_B_PER_W = BATCH // _NW


@functools.partial(
    pl.kernel,
    mesh=plsc.VectorSubcoreMesh(core_axis_name="c", subcore_axis_name="s"),
    out_type=jax.ShapeDtypeStruct((BATCH, EMBED_DIM), jnp.float32),
    scratch_types=[
        pltpu.VMEM((_B_PER_W,), jnp.int32),
        pltpu.VMEM((_B_PER_W, EMBED_DIM), jnp.float32),
        pltpu.SemaphoreType.DMA,
    ],
)
def _gather_kernel(table_hbm, idx_hbm, out_hbm, idx_v, rows_v, sem):
    wid = lax.axis_index("s") * _NC + lax.axis_index("c")
    base = wid * _B_PER_W
    pltpu.sync_copy(idx_hbm.at[pl.ds(base, _B_PER_W)], idx_v)

    def fire(g, carry):
        vec = idx_v[pl.ds(g * _L, _L)]
        for t in range(_L):
            i = vec[t]
            pltpu.make_async_copy(
                table_hbm.at[i], rows_v.at[g * _L + t], sem
            ).start()
        return carry

    lax.fori_loop(0, _B_PER_W // _L, fire, 0)
    pltpu.make_async_copy(
        table_hbm.at[pl.ds(0, _B_PER_W)], rows_v, sem
    ).wait()
    pltpu.sync_copy(rows_v, out_hbm.at[pl.ds(base, _B_PER_W)])


def kernel(node_indices, embeddings):
    idx = node_indices.astype(jnp.int32)
    return _gather_kernel(embeddings, idx)
